# Initial kernel scaffold; baseline (speedup 1.0000x reference)
import jax
import jax.numpy as jnp
from jax.experimental import pallas as pl


def kernel(x, node_index, edge_index, edge_attr, node_features, W_nf, b_nf, W_edge, b_edge, Wg, bg, ln_g, ln_b, t, W_pred, b_pred):
    raise NotImplementedError("write your pallas kernel here")


# speedup vs baseline: 1.9863x; 1.9863x over previous
import jax
import jax.numpy as jnp
from jax.experimental import pallas as pl

MSG_EPS = 1e-7


def _layer_norm(h, g, b):
    mu = jnp.mean(h, axis=-1, keepdims=True)
    var = jnp.var(h, axis=-1, keepdims=True)
    return g * (h - mu) / jnp.sqrt(var + 1e-5) + b


def _gen_conv(h, src, dst, edge_emb, W, b, t, n):
    msg = jax.nn.relu(h[src] + edge_emb) + MSG_EPS
    p = jnp.exp(msg * t)
    s1 = jax.ops.segment_sum(p, dst, num_segments=n)
    s2 = jax.ops.segment_sum(msg * p, dst, num_segments=n)
    m = s2 / (s1 + 1e-16)
    out = h + m
    return out @ W + b


def _pred_kernel(h_ref, w_ref, b_ref, o_ref):
    o_ref[...] = h_ref[...] @ w_ref[...] + b_ref[...]


def kernel(x, node_index, edge_index, edge_attr, node_features, W_nf, b_nf, W_edge, b_edge, Wg, bg, ln_g, ln_b, t, W_pred, b_pred):
    n = node_index.shape[0]
    src = edge_index[0]
    dst = edge_index[1]
    nf = node_features[node_index]
    h = nf @ W_nf + b_nf
    edge_emb = edge_attr @ W_edge + b_edge
    h = _gen_conv(h, src, dst, edge_emb, Wg[0], bg[0], t[0], n)
    for layer in range(1, 7):
        h1 = _layer_norm(h, ln_g[layer - 1], ln_b[layer - 1])
        h2 = jax.nn.relu(h1)
        h = _gen_conv(h2, src, dst, edge_emb, Wg[layer], bg[layer], t[layer], n) + h
    h = jax.nn.relu(_layer_norm(h, ln_g[6], ln_b[6]))
    ntasks = W_pred.shape[1]
    return pl.pallas_call(
        _pred_kernel,
        grid=(n // 2000,),
        in_specs=[
            pl.BlockSpec((2000, 128), lambda i: (i, 0)),
            pl.BlockSpec((128, ntasks), lambda i: (0, 0)),
            pl.BlockSpec((ntasks,), lambda i: (0,)),
        ],
        out_specs=pl.BlockSpec((2000, ntasks), lambda i: (i, 0)),
        out_shape=jax.ShapeDtypeStruct((n, ntasks), jnp.float32),
    )(h, W_pred, b_pred)


# speedup vs baseline: 4.5674x; 2.2995x over previous
import functools

import jax
import jax.numpy as jnp
from jax import lax
from jax.experimental import pallas as pl
from jax.experimental.pallas import tpu as pltpu
from jax.experimental.pallas import tpu_sc as plsc

MSG_EPS = 1e-7
N_NODES = 10000
N_EDGES = 320000
HIDDEN = 128
NUM_LAYERS = 7

EB = 80
ROWS_A = 624
ROWS_REM = N_NODES - 16 * ROWS_A
ZROWS = 104

_MESH = plsc.VectorSubcoreMesh(
    core_axis_name="c", subcore_axis_name="s", num_cores=2, num_subcores=16)

_HI = jax.lax.Precision.HIGHEST


def _msg_body(g_hbm, emb_hbm, src_hbm, dst_hbm, t_hbm,
              out_hbm, acc, srcv, dstv, gv, ev, outv, zv, tv, sem):
    c = lax.axis_index("c")
    s = lax.axis_index("s")

    zero16 = jnp.zeros((16,), jnp.float32)

    def zrow(j, carry):
        for q in range(8):
            zv[j, pl.ds(q * 16, 16)] = zero16
        return carry

    lax.fori_loop(0, ZROWS, zrow, 0)

    def zcopy(k, carry):
        pltpu.sync_copy(zv, acc.at[pl.ds(ROWS_A * s + ZROWS * k, ZROWS)])
        return carry

    lax.fori_loop(0, ROWS_A // ZROWS, zcopy, 0)

    @pl.when(s == 0)
    def _():
        pltpu.sync_copy(zv.at[pl.ds(0, ROWS_REM)],
                        acc.at[pl.ds(16 * ROWS_A, ROWS_REM)])

    plsc.subcore_barrier()

    pltpu.sync_copy(t_hbm, tv)
    tvec = tv[...]

    n_blocks = N_EDGES // 16 // EB
    base = s * (N_EDGES // 16)

    def compute(col):
        def edge(j, carry2):
            for q in range(4):
                g16 = gv[j, pl.ds(col + q * 16, 16)]
                e16 = ev[j, pl.ds(col + q * 16, 16)]
                msg = jnp.maximum(g16 + e16, 0.0) + MSG_EPS
                p = jnp.exp(msg * tvec)
                outv[j, pl.ds(q * 16, 16)] = p
                outv[j, pl.ds(64 + q * 16, 16)] = msg * p
            return carry2

        lax.fori_loop(0, EB, edge, 0)

    def blk(i, carry):
        off = base + i * EB
        pltpu.sync_copy(src_hbm.at[pl.ds(off, EB)], srcv)
        pltpu.sync_copy(dst_hbm.at[pl.ds(off, EB)], dstv)
        pltpu.sync_copy(emb_hbm.at[pl.ds(off, EB)], ev)
        pltpu.async_copy(g_hbm.at[srcv], gv, sem).wait()

        @pl.when(c == 0)
        def _():
            compute(0)

        @pl.when(c == 1)
        def _():
            compute(64)

        pltpu.sync_copy(outv, acc.at[dstv], add=True)
        return carry

    lax.fori_loop(0, n_blocks, blk, 0)
    plsc.subcore_barrier()

    pltpu.sync_copy(acc.at[pl.ds(ROWS_A * s, ROWS_A)],
                    out_hbm.at[c, pl.ds(ROWS_A * s, ROWS_A)])

    @pl.when(s == 0)
    def _():
        pltpu.sync_copy(acc.at[pl.ds(16 * ROWS_A, ROWS_REM)],
                        out_hbm.at[c, pl.ds(16 * ROWS_A, ROWS_REM)])


_msg_kernel = functools.partial(
    pl.kernel,
    out_type=jax.ShapeDtypeStruct((2, N_NODES, HIDDEN), jnp.float32),
    mesh=_MESH,
    scratch_types=[
        pltpu.VMEM_SHARED((N_NODES, HIDDEN), jnp.float32),
        pltpu.VMEM((EB,), jnp.int32),
        pltpu.VMEM((EB,), jnp.int32),
        pltpu.VMEM((EB, HIDDEN), jnp.float32),
        pltpu.VMEM((EB, HIDDEN), jnp.float32),
        pltpu.VMEM((EB, HIDDEN), jnp.float32),
        pltpu.VMEM((ZROWS, HIDDEN), jnp.float32),
        pltpu.VMEM((16,), jnp.float32),
        pltpu.SemaphoreType.DMA,
    ],
)(_msg_body)


def _nf_body(tab_hbm, idx_hbm, out_hbm, idxv, rowsv, sem):
    c = lax.axis_index("c")
    s = lax.axis_index("s")
    w = s * 2 + c
    n_blocks = N_NODES // EB

    def blk(k, carry):
        bid = w + 32 * k

        @pl.when(bid < n_blocks)
        def _():
            pltpu.sync_copy(idx_hbm.at[pl.ds(bid * EB, EB)], idxv)
            pltpu.async_copy(tab_hbm.at[idxv], rowsv, sem).wait()
            pltpu.sync_copy(rowsv, out_hbm.at[pl.ds(bid * EB, EB)])
        return carry

    lax.fori_loop(0, (n_blocks + 31) // 32, blk, 0)


_nf_kernel = functools.partial(
    pl.kernel,
    out_type=jax.ShapeDtypeStruct((N_NODES, HIDDEN), jnp.float32),
    mesh=_MESH,
    scratch_types=[
        pltpu.VMEM((EB,), jnp.int32),
        pltpu.VMEM((EB, HIDDEN), jnp.float32),
        pltpu.SemaphoreType.DMA,
    ],
)(_nf_body)


def _mm_tc(a_ref, w_ref, b_ref, o_ref):
    o_ref[...] = lax.dot(a_ref[...], w_ref[...], precision=_HI) + b_ref[...]


def _update_tc(s_ref, g_ref, h_ref, w_ref, b_ref, lng_ref, lnb_ref,
               hout_ref, gout_ref, *, with_res):
    s0 = s_ref[0]
    s1 = s_ref[1]
    m = jnp.concatenate(
        [s0[:, 64:] / (s0[:, :64] + 1e-16),
         s1[:, 64:] / (s1[:, :64] + 1e-16)], axis=1)
    out = lax.dot(g_ref[...] + m, w_ref[...], precision=_HI) + b_ref[...]
    if with_res:
        out = out + h_ref[...]
    hout_ref[...] = out
    mu = jnp.mean(out, axis=1, keepdims=True)
    var = jnp.mean((out - mu) ** 2, axis=1, keepdims=True)
    gn = lng_ref[...] * (out - mu) / jnp.sqrt(var + 1e-5) + lnb_ref[...]
    gout_ref[...] = jnp.maximum(gn, 0.0)


def _row_spec(bn, width):
    return pl.BlockSpec((bn, width), lambda i: (i, 0))


def _full_spec(shape):
    nd = len(shape)
    return pl.BlockSpec(shape, lambda i: (0,) * nd)


def kernel(x, node_index, edge_index, edge_attr, node_features, W_nf, b_nf,
           W_edge, b_edge, Wg, bg, ln_g, ln_b, t, W_pred, b_pred):
    del x
    n, e, hdim = N_NODES, N_EDGES, HIDDEN
    ntasks = W_pred.shape[1]
    src = edge_index[0].astype(jnp.int32)
    dst = edge_index[1].astype(jnp.int32)
    node_index = node_index.astype(jnp.int32)

    tab128 = jnp.pad(node_features, ((0, 0), (0, hdim - 8)))
    nf = _nf_kernel(tab128, node_index)
    W128 = jnp.pad(W_nf, ((0, hdim - 8), (0, 0)))

    bn = 2000
    grid = (n // bn,)
    h = pl.pallas_call(
        _mm_tc,
        grid=grid,
        in_specs=[_row_spec(bn, hdim), _full_spec((hdim, hdim)),
                  _full_spec((1, hdim))],
        out_specs=_row_spec(bn, hdim),
        out_shape=jax.ShapeDtypeStruct((n, hdim), jnp.float32),
    )(nf, W128, b_nf.reshape(1, hdim))

    eb = 4000
    emb = pl.pallas_call(
        _mm_tc,
        grid=(e // eb,),
        in_specs=[_row_spec(eb, 8), _full_spec((8, hdim)),
                  _full_spec((1, hdim))],
        out_specs=_row_spec(eb, hdim),
        out_shape=jax.ShapeDtypeStruct((e, hdim), jnp.float32),
    )(edge_attr, W_edge, b_edge.reshape(1, hdim))

    g = h
    for layer in range(NUM_LAYERS):
        t16 = jnp.broadcast_to(t[layer], (16,)).astype(jnp.float32)
        S = _msg_kernel(g, emb, src, dst, t16)
        h, g = pl.pallas_call(
            functools.partial(_update_tc, with_res=layer > 0),
            grid=grid,
            in_specs=[pl.BlockSpec((2, bn, hdim), lambda i: (0, i, 0)),
                      _row_spec(bn, hdim), _row_spec(bn, hdim),
                      _full_spec((hdim, hdim)), _full_spec((1, hdim)),
                      _full_spec((1, hdim)), _full_spec((1, hdim))],
            out_specs=[_row_spec(bn, hdim), _row_spec(bn, hdim)],
            out_shape=[jax.ShapeDtypeStruct((n, hdim), jnp.float32),
                       jax.ShapeDtypeStruct((n, hdim), jnp.float32)],
        )(S, g, h, Wg[layer], bg[layer].reshape(1, hdim),
          ln_g[layer].reshape(1, hdim), ln_b[layer].reshape(1, hdim))

    return pl.pallas_call(
        _mm_tc,
        grid=grid,
        in_specs=[_row_spec(bn, hdim), _full_spec((hdim, ntasks)),
                  _full_spec((1, ntasks))],
        out_specs=_row_spec(bn, ntasks),
        out_shape=jax.ShapeDtypeStruct((n, ntasks), jnp.float32),
    )(g, W_pred, b_pred.reshape(1, ntasks))
